# Initial kernel scaffold; baseline (speedup 1.0000x reference)
import jax
import jax.numpy as jnp
from jax.experimental import pallas as pl


def kernel(card_indices, embedding_table):
    raise NotImplementedError("write your pallas kernel here")


# speedup vs baseline: 2.3065x; 2.3065x over previous
import functools

import jax
import jax.numpy as jnp
from jax import lax
from jax.experimental import pallas as pl
from jax.experimental.pallas import tpu as pltpu
from jax.experimental.pallas import tpu_sc as plsc

VOCAB = 53
EMBED_DIM = 32
BATCH, SEQ = 16384, 200
TOTAL = BATCH * SEQ
NUM_WORKERS = 32
PER_W = TOTAL // NUM_WORKERS
CHUNK = 1024
NUM_CHUNKS = PER_W // CHUNK
GROUPS = CHUNK // 16
TAB = VOCAB * EMBED_DIM

_mesh = plsc.VectorSubcoreMesh(core_axis_name="c", subcore_axis_name="s")


@functools.partial(
    pl.kernel,
    mesh=_mesh,
    out_type=jax.ShapeDtypeStruct((TOTAL * EMBED_DIM,), jnp.float32),
    scratch_types=[
        pltpu.VMEM((TAB,), jnp.float32),
        pltpu.VMEM((CHUNK,), jnp.int32),
        pltpu.VMEM((CHUNK * EMBED_DIM,), jnp.float32),
    ],
    compiler_params=pltpu.CompilerParams(needs_layout_passes=False),
)
def _embed_sc(idx_hbm, table_hbm, out_hbm, tab_v, idx_v, rows_v):
    wid = lax.axis_index("s") * 2 + lax.axis_index("c")
    pltpu.sync_copy(table_hbm, tab_v)
    lane = lax.iota(jnp.int32, 16)
    row_pos = lane * EMBED_DIM

    def chunk_body(g, carry):
        base = wid * PER_W + g * CHUNK
        pltpu.sync_copy(idx_hbm.at[pl.ds(base, CHUNK)], idx_v)

        def group_body(i, c):
            iv = idx_v[pl.ds(i * 16, 16)]
            src = iv * EMBED_DIM
            dst = i * (16 * EMBED_DIM) + row_pos
            for d in range(EMBED_DIM):
                vals = plsc.load_gather(tab_v, [src + d])
                plsc.store_scatter(rows_v, [dst + d], vals)
            return c

        lax.fori_loop(0, GROUPS, group_body, 0)
        pltpu.sync_copy(rows_v, out_hbm.at[pl.ds(base * EMBED_DIM, CHUNK * EMBED_DIM)])
        return carry

    lax.fori_loop(0, NUM_CHUNKS, chunk_body, 0)


def kernel(card_indices, embedding_table):
    idx_flat = card_indices.astype(jnp.int32).reshape(TOTAL)
    tab_flat = embedding_table.reshape(TAB)
    out = _embed_sc(idx_flat, tab_flat)
    return out.reshape(BATCH, SEQ, EMBED_DIM)


# speedup vs baseline: 5.4039x; 2.3429x over previous
import functools

import jax
import jax.numpy as jnp
from jax import lax
from jax.experimental import pallas as pl
from jax.experimental.pallas import tpu as pltpu
from jax.experimental.pallas import tpu_sc as plsc

VOCAB = 53
EMBED_DIM = 32
BATCH, SEQ = 16384, 200
TOTAL = BATCH * SEQ
NUM_WORKERS = 32
PER_W = TOTAL // NUM_WORKERS
CHUNK = 1024
NUM_CHUNKS = PER_W // CHUNK
GROUPS = CHUNK // 16
TAB = VOCAB * EMBED_DIM

_mesh = plsc.VectorSubcoreMesh(core_axis_name="c", subcore_axis_name="s")


@functools.partial(
    pl.kernel,
    mesh=_mesh,
    out_type=jax.ShapeDtypeStruct((TOTAL * EMBED_DIM,), jnp.float32),
    scratch_types=[
        pltpu.VMEM((TAB,), jnp.float32),
        pltpu.VMEM((CHUNK,), jnp.int32),
        pltpu.VMEM((CHUNK * EMBED_DIM,), jnp.float32),
    ],
    compiler_params=pltpu.CompilerParams(
        needs_layout_passes=False, disable_bounds_checks=True
    ),
)
def _embed_sc(idx_hbm, table_hbm, out_hbm, tab_v, idx_v, rows_v):
    wid = lax.axis_index("s") * 2 + lax.axis_index("c")
    pltpu.sync_copy(table_hbm, tab_v)
    lane = lax.iota(jnp.int32, 16)

    def chunk_body(g, carry):
        base = wid * PER_W + g * CHUNK
        pltpu.sync_copy(idx_hbm.at[pl.ds(base, CHUNK)], idx_v)

        def group_body(i, c):
            iv = idx_v[pl.ds(i * 16, 16)]
            src = iv * EMBED_DIM
            out_base = i * (16 * EMBED_DIM)
            for r in range(16):
                sel = jnp.full((16,), r, jnp.int32)
                bc = jnp.take_along_axis(src, sel, axis=0)
                a0 = bc + lane
                v0 = plsc.load_gather(tab_v, [a0])
                v1 = plsc.load_gather(tab_v, [a0 + 16])
                rows_v[pl.ds(out_base + r * EMBED_DIM, 16)] = v0
                rows_v[pl.ds(out_base + r * EMBED_DIM + 16, 16)] = v1
            return c

        lax.fori_loop(0, GROUPS, group_body, 0)
        pltpu.sync_copy(rows_v, out_hbm.at[pl.ds(base * EMBED_DIM, CHUNK * EMBED_DIM)])
        return carry

    lax.fori_loop(0, NUM_CHUNKS, chunk_body, 0)


def kernel(card_indices, embedding_table):
    idx_flat = card_indices.astype(jnp.int32).reshape(TOTAL)
    tab_flat = embedding_table.reshape(TAB)
    out = _embed_sc(idx_flat, tab_flat)
    return out.reshape(BATCH, SEQ, EMBED_DIM)


# speedup vs baseline: 5.5892x; 1.0343x over previous
import functools

import jax
import jax.numpy as jnp
from jax import lax
from jax.experimental import pallas as pl
from jax.experimental.pallas import tpu as pltpu
from jax.experimental.pallas import tpu_sc as plsc

VOCAB = 53
EMBED_DIM = 32
BATCH, SEQ = 16384, 200
TOTAL = BATCH * SEQ
NUM_WORKERS = 32
PER_W = TOTAL // NUM_WORKERS
CHUNK = 1024
NUM_CHUNKS = PER_W // CHUNK
GROUPS = CHUNK // 16
TAB = VOCAB * EMBED_DIM

_mesh = plsc.VectorSubcoreMesh(core_axis_name="c", subcore_axis_name="s")


@functools.partial(
    pl.kernel,
    mesh=_mesh,
    out_type=jax.ShapeDtypeStruct((TOTAL * EMBED_DIM,), jnp.float32),
    scratch_types=[
        pltpu.VMEM((TAB,), jnp.float32),
        pltpu.VMEM((4, CHUNK), jnp.int32),
        pltpu.VMEM((2, CHUNK * EMBED_DIM), jnp.float32),
        pltpu.SemaphoreType.DMA,
        pltpu.SemaphoreType.DMA,
        pltpu.SemaphoreType.DMA,
        pltpu.SemaphoreType.DMA,
        pltpu.SemaphoreType.DMA,
        pltpu.SemaphoreType.DMA,
    ],
    compiler_params=pltpu.CompilerParams(
        needs_layout_passes=False, disable_bounds_checks=True
    ),
)
def _embed_sc(
    idx_hbm, table_hbm, out_hbm, tab_v, idx_v, rows_v,
    sin0, sin1, sin2, sin3, sout0, sout1,
):
    wid = lax.axis_index("s") * 2 + lax.axis_index("c")
    pltpu.sync_copy(table_hbm, tab_v)
    lane = lax.iota(jnp.int32, 16)
    sins = (sin0, sin1, sin2, sin3)
    souts = (sout0, sout1)
    base0 = wid * PER_W

    def in_slice(g):
        return idx_hbm.at[pl.ds(base0 + g * CHUNK, CHUNK)]

    def out_slice(g):
        return out_hbm.at[
            pl.ds((base0 + g * CHUNK) * EMBED_DIM, CHUNK * EMBED_DIM)
        ]

    for b in range(3):
        pltpu.async_copy(in_slice(b), idx_v.at[b], sins[b])

    def compute(ib, rb):
        def group_body(i, c):
            iv = idx_v[ib, pl.ds(i * 16, 16)]
            src = iv * EMBED_DIM
            out_base = i * (16 * EMBED_DIM)
            for r in range(16):
                sel = jnp.full((16,), r, jnp.int32)
                bc = jnp.take_along_axis(src, sel, axis=0)
                a0 = bc + lane
                v0 = plsc.load_gather(tab_v, [a0])
                v1 = plsc.load_gather(tab_v, [a0 + 16])
                rows_v[rb, pl.ds(out_base + r * EMBED_DIM, 16)] = v0
                rows_v[rb, pl.ds(out_base + r * EMBED_DIM + 16, 16)] = v1
            return c

        lax.fori_loop(0, GROUPS, group_body, 0)

    def super_body(j, carry):
        for b in range(4):
            g = 4 * j + b
            rb = b % 2
            pltpu.make_async_copy(in_slice(g), idx_v.at[b], sins[b]).wait()
            @pl.when(g + 3 < NUM_CHUNKS)
            def _():
                pltpu.async_copy(
                    in_slice(g + 3), idx_v.at[(b + 3) % 4], sins[(b + 3) % 4]
                )
            @pl.when(g >= 2)
            def _():
                pltpu.make_async_copy(
                    rows_v.at[rb], out_slice(0), souts[rb]
                ).wait()
            compute(b, rb)
            pltpu.async_copy(rows_v.at[rb], out_slice(g), souts[rb])
        return carry

    lax.fori_loop(0, NUM_CHUNKS // 4, super_body, 0)
    pltpu.make_async_copy(rows_v.at[0], out_slice(0), souts[0]).wait()
    pltpu.make_async_copy(rows_v.at[1], out_slice(0), souts[1]).wait()


def kernel(card_indices, embedding_table):
    idx_flat = card_indices.astype(jnp.int32).reshape(TOTAL)
    tab_flat = embedding_table.reshape(TAB)
    out = _embed_sc(idx_flat, tab_flat)
    return out.reshape(BATCH, SEQ, EMBED_DIM)
